# baseline (speedup 1.0000x reference)
import functools

import jax
import jax.numpy as jnp
from jax import lax
from jax.experimental import pallas as pl
from jax.experimental.pallas import tpu as pltpu
from jax.experimental.pallas import tpu_sc as plsc

N_EMBED = 8192
E_DIM = 64
BETA = 0.25

BM = 512
BN = 1024
NHALF = 4

_SC_CORES = 2
_SC_SUBCORES = 16
_NW = _SC_CORES * _SC_SUBCORES

_BIG = jnp.float32(1e4)


def _vq_body(z_ref, wt_ref, zsq_ref, wsq_ref,
             idx0_ref, idx1_ref, x0_ref, x1_ref, loss_ref,
             mind_ref, arg_ref, x_ref, minx_ref, *, n_rows):
    j = pl.program_id(1)
    nj = pl.num_programs(1)

    @pl.when((j == 0) | (j == NHALF))
    def _init_half():
        mind_ref[...] = jnp.full((BM, 1), jnp.inf, dtype=jnp.float32)
        arg_ref[...] = jnp.zeros((BM, 1), dtype=jnp.int32)
        x_ref[...] = jnp.zeros((BM, 1), dtype=jnp.float32)

    @pl.when(j == 0)
    def _init_row():
        minx_ref[...] = jnp.full((BM, 1), jnp.inf, dtype=jnp.float32)

    mm = jnp.dot(z_ref[...], wt_ref[...], preferred_element_type=jnp.float32)
    d2 = jnp.maximum(zsq_ref[...] + wsq_ref[...] - 2.0 * mm, 0.0)
    dd = jnp.sqrt(d2)

    tmin = jnp.min(dd, axis=1, keepdims=True)
    iota = lax.broadcasted_iota(jnp.int32, (BM, BN), 1)
    tloc = jnp.min(jnp.where(dd == tmin, iota, BN), axis=1, keepdims=True)
    tx = jnp.min(jnp.where(iota == tloc, d2, jnp.inf), axis=1, keepdims=True)
    targ = tloc + j * BN

    upd = tmin < mind_ref[...]
    mind_ref[...] = jnp.where(upd, tmin, mind_ref[...])
    arg_ref[...] = jnp.where(upd, targ, arg_ref[...])
    x_ref[...] = jnp.where(upd, tx, x_ref[...])
    minx_ref[...] = jnp.minimum(minx_ref[...],
                                jnp.min(d2, axis=1, keepdims=True))

    @pl.when(j == NHALF - 1)
    def _half0_out():
        idx0_ref[...] = arg_ref[...]
        x0_ref[...] = x_ref[...]

    @pl.when(j == nj - 1)
    def _half1_out():
        idx1_ref[...] = arg_ref[...]
        x1_ref[...] = x_ref[...]
        part = jnp.sum(minx_ref[...]) * ((1.0 + BETA) / (n_rows * E_DIM))

        @pl.when(pl.program_id(0) == 0)
        def _zero():
            loss_ref[0, 0] = 0.0

        loss_ref[0, 0] += part


def _distance_halves(z_bf, wt, z_sq, w_sq):
    m = z_bf.shape[0]
    grid = (m // BM, N_EMBED // BN)
    return pl.pallas_call(
        functools.partial(_vq_body, n_rows=m),
        grid=grid,
        in_specs=[
            pl.BlockSpec((BM, E_DIM), lambda i, j: (i, 0)),
            pl.BlockSpec((E_DIM, BN), lambda i, j: (0, j)),
            pl.BlockSpec((BM, 1), lambda i, j: (i, 0)),
            pl.BlockSpec((1, BN), lambda i, j: (0, j)),
        ],
        out_specs=[
            pl.BlockSpec((BM, 1), lambda i, j: (i, 0)),
            pl.BlockSpec((BM, 1), lambda i, j: (i, 0)),
            pl.BlockSpec((BM, 1), lambda i, j: (i, 0)),
            pl.BlockSpec((BM, 1), lambda i, j: (i, 0)),
            pl.BlockSpec(memory_space=pltpu.SMEM, block_shape=(1, 1),
                         index_map=lambda i, j: (0, 0)),
        ],
        out_shape=[
            jax.ShapeDtypeStruct((m, 1), jnp.int32),
            jax.ShapeDtypeStruct((m, 1), jnp.int32),
            jax.ShapeDtypeStruct((m, 1), jnp.float32),
            jax.ShapeDtypeStruct((m, 1), jnp.float32),
            jax.ShapeDtypeStruct((1, 1), jnp.float32),
        ],
        scratch_shapes=[
            pltpu.VMEM((BM, 1), jnp.float32),
            pltpu.VMEM((BM, 1), jnp.int32),
            pltpu.VMEM((BM, 1), jnp.float32),
            pltpu.VMEM((BM, 1), jnp.float32),
        ],
    )(z_bf, wt, z_sq, w_sq)


def _rn_bf16(x):
    u = lax.bitcast_convert_type(x, jnp.uint32)
    r = u + jnp.uint32(0x7FFF) + ((u >> 16) & jnp.uint32(1))
    return lax.bitcast_convert_type(r & jnp.uint32(0xFFFF0000), jnp.float32)


def _bf16_split3(x):
    a = _rn_bf16(x)
    r1 = x - a
    b = _rn_bf16(r1)
    c = r1 - b
    return (a.astype(jnp.bfloat16), b.astype(jnp.bfloat16),
            c.astype(jnp.bfloat16))


def _cross_half_pick(x0, x1):
    m = x0.shape[0]
    a0, b0, c0 = _bf16_split3(x0 * jnp.float32(-0.5))
    a1, b1, c1 = _bf16_split3(x1 * jnp.float32(-0.5))
    A = jnp.concatenate(
        [jnp.stack([a0, b0, c0, a1, b1, c1], axis=1),
         jnp.zeros((m, E_DIM - 6), jnp.bfloat16)], axis=1)
    B = jnp.zeros((N_EMBED, E_DIM), jnp.float32)
    B = B.at[0, 0:3].set(1.0).at[N_EMBED // 2, 3:6].set(1.0)
    ws = jnp.full((N_EMBED,), _BIG).at[0].set(0.0).at[N_EMBED // 2].set(0.0)
    zs = jnp.zeros((m,), jnp.float32)
    A, B, zs, ws = lax.optimization_barrier((A, B, zs, ws))
    mm = lax.dot_general(A, B, (((1,), (1,)), ((), ())),
                         preferred_element_type=jnp.float32)
    xr = jnp.maximum(zs[:, None] + ws[None, :] - 2.0 * mm, 0.0)
    pos = jnp.argmin(jnp.sqrt(xr), axis=1)
    return pos != 0


def _codebook_gather(table, idx):
    b = idx.shape[0]
    b_per_w = b // _NW
    mesh = plsc.VectorSubcoreMesh(core_axis_name="c", subcore_axis_name="s")

    @functools.partial(
        pl.kernel,
        mesh=mesh,
        out_type=jax.ShapeDtypeStruct((b, E_DIM), jnp.float32),
        scratch_types=[
            pltpu.VMEM((b_per_w,), jnp.int32),
            pltpu.VMEM((b_per_w, E_DIM), jnp.float32),
            pltpu.SemaphoreType.DMA,
        ],
        compiler_params=pltpu.CompilerParams(use_tc_tiling_on_sc=False),
    )
    def gather_kernel(table_hbm, idx_hbm, out_hbm, idx_v, rows_v, sem):
        wid = lax.axis_index("s") * _SC_CORES + lax.axis_index("c")
        base = wid * b_per_w
        pltpu.sync_copy(idx_hbm.at[pl.ds(base, b_per_w)], idx_v)
        pltpu.async_copy(table_hbm.at[idx_v], rows_v, sem).wait()
        pltpu.sync_copy(rows_v, out_hbm.at[pl.ds(base, b_per_w)])

    return gather_kernel(table, idx)


def kernel(z, embedding_weight):
    z_flat = z.reshape(-1, E_DIM)
    z_sq = jnp.sum(z_flat * z_flat, axis=1, keepdims=True)
    w_sq = jnp.sum(embedding_weight * embedding_weight, axis=1)[None, :]
    wt = embedding_weight.T
    z_bf = z_flat.astype(jnp.bfloat16).astype(jnp.float32)

    idx0, idx1, x0, x1, loss = _distance_halves(z_bf, wt, z_sq, w_sq)
    pick1 = _cross_half_pick(x0.reshape(-1), x1.reshape(-1))
    min_encoding_indices = jnp.where(pick1, idx1.reshape(-1),
                                     idx0.reshape(-1))

    z_q = _codebook_gather(embedding_weight, min_encoding_indices)
    z_q = z_q.reshape(z.shape)

    z_q_st = z + lax.stop_gradient(z_q - z)
    return (z_q_st, loss.reshape(()), min_encoding_indices)


# speedup vs baseline: 1.0244x; 1.0244x over previous
import functools

import jax
import jax.numpy as jnp
from jax import lax
from jax.experimental import pallas as pl
from jax.experimental.pallas import tpu as pltpu
from jax.experimental.pallas import tpu_sc as plsc

N_EMBED = 8192
E_DIM = 64
BETA = 0.25

BM = 512
BN = 1024
NHALF = 4

_SC_CORES = 2
_SC_SUBCORES = 16
_NW = _SC_CORES * _SC_SUBCORES

_BIG = jnp.float32(1e4)


def _vq_body(z_ref, wt_ref, zsq_ref, wsq_ref,
             idx0_ref, idx1_ref, x0_ref, x1_ref, loss_ref,
             mind_ref, arg_ref, x_ref, *, n_rows):
    j = pl.program_id(1)
    nj = pl.num_programs(1)

    @pl.when((j == 0) | (j == NHALF))
    def _init_half():
        mind_ref[...] = jnp.full((BM, 1), jnp.inf, dtype=jnp.float32)
        arg_ref[...] = jnp.zeros((BM, 1), dtype=jnp.int32)
        x_ref[...] = jnp.zeros((BM, 1), dtype=jnp.float32)

    mm = jnp.dot(z_ref[...], wt_ref[...], preferred_element_type=jnp.float32)
    d2 = jnp.maximum(zsq_ref[...] + wsq_ref[...] - 2.0 * mm, 0.0)
    dd = jnp.sqrt(d2)

    tmin = jnp.min(dd, axis=1, keepdims=True)
    iota = lax.broadcasted_iota(jnp.int32, (BM, BN), 1)
    tloc = jnp.min(jnp.where(dd == tmin, iota, BN), axis=1, keepdims=True)
    tx = jnp.min(jnp.where(iota == tloc, d2, jnp.inf), axis=1, keepdims=True)
    targ = tloc + j * BN

    upd = tmin < mind_ref[...]
    mind_ref[...] = jnp.where(upd, tmin, mind_ref[...])
    arg_ref[...] = jnp.where(upd, targ, arg_ref[...])
    x_ref[...] = jnp.where(upd, tx, x_ref[...])

    @pl.when(j == NHALF - 1)
    def _half0_out():
        idx0_ref[...] = arg_ref[...]
        x0_ref[...] = x_ref[...]

    @pl.when(j == nj - 1)
    def _half1_out():
        idx1_ref[...] = arg_ref[...]
        x1_ref[...] = x_ref[...]
        part = (jnp.sum(jnp.minimum(x0_ref[...], x_ref[...]))
                * ((1.0 + BETA) / (n_rows * E_DIM)))

        @pl.when(pl.program_id(0) == 0)
        def _zero():
            loss_ref[0, 0] = 0.0

        loss_ref[0, 0] += part


def _distance_halves(z_bf, wt, z_sq, w_sq):
    m = z_bf.shape[0]
    grid = (m // BM, N_EMBED // BN)
    return pl.pallas_call(
        functools.partial(_vq_body, n_rows=m),
        grid=grid,
        in_specs=[
            pl.BlockSpec((BM, E_DIM), lambda i, j: (i, 0)),
            pl.BlockSpec((E_DIM, BN), lambda i, j: (0, j)),
            pl.BlockSpec((BM, 1), lambda i, j: (i, 0)),
            pl.BlockSpec((1, BN), lambda i, j: (0, j)),
        ],
        out_specs=[
            pl.BlockSpec((BM, 1), lambda i, j: (i, 0)),
            pl.BlockSpec((BM, 1), lambda i, j: (i, 0)),
            pl.BlockSpec((BM, 1), lambda i, j: (i, 0)),
            pl.BlockSpec((BM, 1), lambda i, j: (i, 0)),
            pl.BlockSpec(memory_space=pltpu.SMEM, block_shape=(1, 1),
                         index_map=lambda i, j: (0, 0)),
        ],
        out_shape=[
            jax.ShapeDtypeStruct((m, 1), jnp.int32),
            jax.ShapeDtypeStruct((m, 1), jnp.int32),
            jax.ShapeDtypeStruct((m, 1), jnp.float32),
            jax.ShapeDtypeStruct((m, 1), jnp.float32),
            jax.ShapeDtypeStruct((1, 1), jnp.float32),
        ],
        scratch_shapes=[
            pltpu.VMEM((BM, 1), jnp.float32),
            pltpu.VMEM((BM, 1), jnp.int32),
            pltpu.VMEM((BM, 1), jnp.float32),
        ],
    )(z_bf, wt, z_sq, w_sq)


def _rn_bf16(x):
    u = lax.bitcast_convert_type(x, jnp.uint32)
    r = u + jnp.uint32(0x7FFF) + ((u >> 16) & jnp.uint32(1))
    return lax.bitcast_convert_type(r & jnp.uint32(0xFFFF0000), jnp.float32)


def _bf16_split3(x):
    a = _rn_bf16(x)
    r1 = x - a
    b = _rn_bf16(r1)
    c = r1 - b
    return (a.astype(jnp.bfloat16), b.astype(jnp.bfloat16),
            c.astype(jnp.bfloat16))


def _cross_half_pick(x0, x1):
    m = x0.shape[0]
    a0, b0, c0 = _bf16_split3(x0 * jnp.float32(-0.5))
    a1, b1, c1 = _bf16_split3(x1 * jnp.float32(-0.5))
    A = jnp.concatenate(
        [jnp.stack([a0, b0, c0, a1, b1, c1], axis=1),
         jnp.zeros((m, E_DIM - 6), jnp.bfloat16)], axis=1)
    B = jnp.zeros((N_EMBED, E_DIM), jnp.float32)
    B = B.at[0, 0:3].set(1.0).at[N_EMBED // 2, 3:6].set(1.0)
    ws = jnp.full((N_EMBED,), _BIG).at[0].set(0.0).at[N_EMBED // 2].set(0.0)
    zs = jnp.zeros((m,), jnp.float32)
    A, B, zs, ws = lax.optimization_barrier((A, B, zs, ws))
    mm = lax.dot_general(A, B, (((1,), (1,)), ((), ())),
                         preferred_element_type=jnp.float32)
    xr = jnp.maximum(zs[:, None] + ws[None, :] - 2.0 * mm, 0.0)
    pos = jnp.argmin(jnp.sqrt(xr), axis=1)
    return pos != 0


def _codebook_gather(table, idx):
    b = idx.shape[0]
    b_per_w = b // _NW
    mesh = plsc.VectorSubcoreMesh(core_axis_name="c", subcore_axis_name="s")

    @functools.partial(
        pl.kernel,
        mesh=mesh,
        out_type=jax.ShapeDtypeStruct((b, E_DIM), jnp.float32),
        scratch_types=[
            pltpu.VMEM((b_per_w,), jnp.int32),
            pltpu.VMEM((b_per_w, E_DIM), jnp.float32),
            pltpu.SemaphoreType.DMA,
        ],
        compiler_params=pltpu.CompilerParams(use_tc_tiling_on_sc=False),
    )
    def gather_kernel(table_hbm, idx_hbm, out_hbm, idx_v, rows_v, sem):
        wid = lax.axis_index("s") * _SC_CORES + lax.axis_index("c")
        base = wid * b_per_w
        pltpu.sync_copy(idx_hbm.at[pl.ds(base, b_per_w)], idx_v)
        pltpu.async_copy(table_hbm.at[idx_v], rows_v, sem).wait()
        pltpu.sync_copy(rows_v, out_hbm.at[pl.ds(base, b_per_w)])

    return gather_kernel(table, idx)


def kernel(z, embedding_weight):
    z_flat = z.reshape(-1, E_DIM)
    z_sq = jnp.sum(z_flat * z_flat, axis=1, keepdims=True)
    w_sq = jnp.sum(embedding_weight * embedding_weight, axis=1)[None, :]
    wt = embedding_weight.T
    z_bf = z_flat.astype(jnp.bfloat16).astype(jnp.float32)

    idx0, idx1, x0, x1, loss = _distance_halves(z_bf, wt, z_sq, w_sq)
    pick1 = _cross_half_pick(x0.reshape(-1), x1.reshape(-1))
    min_encoding_indices = jnp.where(pick1, idx1.reshape(-1),
                                     idx0.reshape(-1))

    z_q = _codebook_gather(embedding_weight, min_encoding_indices)
    z_q = z_q.reshape(z.shape)

    z_q_st = z + lax.stop_gradient(z_q - z)
    return (z_q_st, loss.reshape(()), min_encoding_indices)


# speedup vs baseline: 1.0299x; 1.0054x over previous
import functools

import jax
import jax.numpy as jnp
from jax import lax
from jax.experimental import pallas as pl
from jax.experimental.pallas import tpu as pltpu
from jax.experimental.pallas import tpu_sc as plsc

N_EMBED = 8192
E_DIM = 64
BETA = 0.25

BM = 512
BN = 1024
NHALF = 4

_SC_CORES = 2
_SC_SUBCORES = 16
_NW = _SC_CORES * _SC_SUBCORES

_BIG = jnp.float32(1e4)


def _vq_body(z_ref, wt_ref, zsq_ref, wsq_ref,
             idx0_ref, idx1_ref, x0_ref, x1_ref, loss_ref,
             mind_ref, arg_ref, x_ref, *, n_rows):
    j = pl.program_id(1)
    nj = pl.num_programs(1)

    @pl.when((j == 0) | (j == NHALF))
    def _init_half():
        mind_ref[...] = jnp.full((BM, 1), jnp.inf, dtype=jnp.float32)
        arg_ref[...] = jnp.zeros((BM, 1), dtype=jnp.int32)
        x_ref[...] = jnp.zeros((BM, 1), dtype=jnp.float32)

    mm = lax.dot_general(z_ref[...], wt_ref[...], (((1,), (0,)), ((), ())),
                         preferred_element_type=jnp.float32)
    d2 = jnp.maximum(zsq_ref[...] + wsq_ref[...] - 2.0 * mm, 0.0)
    dd = jnp.sqrt(d2)

    tmin = jnp.min(dd, axis=1, keepdims=True)
    iota = lax.broadcasted_iota(jnp.int32, (BM, BN), 1)
    tloc = jnp.min(jnp.where(dd == tmin, iota, BN), axis=1, keepdims=True)
    tx = jnp.min(jnp.where(iota == tloc, d2, jnp.inf), axis=1, keepdims=True)
    targ = tloc + j * BN

    upd = tmin < mind_ref[...]
    mind_ref[...] = jnp.where(upd, tmin, mind_ref[...])
    arg_ref[...] = jnp.where(upd, targ, arg_ref[...])
    x_ref[...] = jnp.where(upd, tx, x_ref[...])

    @pl.when(j == NHALF - 1)
    def _half0_out():
        idx0_ref[...] = arg_ref[...]
        x0_ref[...] = x_ref[...]

    @pl.when(j == nj - 1)
    def _half1_out():
        idx1_ref[...] = arg_ref[...]
        x1_ref[...] = x_ref[...]
        part = (jnp.sum(jnp.minimum(x0_ref[...], x_ref[...]))
                * ((1.0 + BETA) / (n_rows * E_DIM)))

        @pl.when(pl.program_id(0) == 0)
        def _zero():
            loss_ref[0, 0] = 0.0

        loss_ref[0, 0] += part


def _distance_halves(z_bf, wt, z_sq, w_sq):
    m = z_bf.shape[0]
    grid = (m // BM, N_EMBED // BN)
    return pl.pallas_call(
        functools.partial(_vq_body, n_rows=m),
        grid=grid,
        in_specs=[
            pl.BlockSpec((BM, E_DIM), lambda i, j: (i, 0)),
            pl.BlockSpec((E_DIM, BN), lambda i, j: (0, j)),
            pl.BlockSpec((BM, 1), lambda i, j: (i, 0)),
            pl.BlockSpec((1, BN), lambda i, j: (0, j)),
        ],
        out_specs=[
            pl.BlockSpec((BM, 1), lambda i, j: (i, 0)),
            pl.BlockSpec((BM, 1), lambda i, j: (i, 0)),
            pl.BlockSpec((BM, 1), lambda i, j: (i, 0)),
            pl.BlockSpec((BM, 1), lambda i, j: (i, 0)),
            pl.BlockSpec(memory_space=pltpu.SMEM, block_shape=(1, 1),
                         index_map=lambda i, j: (0, 0)),
        ],
        out_shape=[
            jax.ShapeDtypeStruct((m, 1), jnp.int32),
            jax.ShapeDtypeStruct((m, 1), jnp.int32),
            jax.ShapeDtypeStruct((m, 1), jnp.float32),
            jax.ShapeDtypeStruct((m, 1), jnp.float32),
            jax.ShapeDtypeStruct((1, 1), jnp.float32),
        ],
        scratch_shapes=[
            pltpu.VMEM((BM, 1), jnp.float32),
            pltpu.VMEM((BM, 1), jnp.int32),
            pltpu.VMEM((BM, 1), jnp.float32),
        ],
    )(z_bf, wt, z_sq, w_sq)


def _rn_bf16(x):
    u = lax.bitcast_convert_type(x, jnp.uint32)
    r = u + jnp.uint32(0x7FFF) + ((u >> 16) & jnp.uint32(1))
    return lax.bitcast_convert_type(r & jnp.uint32(0xFFFF0000), jnp.float32)


def _bf16_split3(x):
    a = _rn_bf16(x)
    r1 = x - a
    b = _rn_bf16(r1)
    c = r1 - b
    return (a.astype(jnp.bfloat16), b.astype(jnp.bfloat16),
            c.astype(jnp.bfloat16))


def _cross_half_pick(x0, x1):
    m = x0.shape[0]
    a0, b0, c0 = _bf16_split3(x0 * jnp.float32(-0.5))
    a1, b1, c1 = _bf16_split3(x1 * jnp.float32(-0.5))
    A = jnp.concatenate(
        [jnp.stack([a0, b0, c0, a1, b1, c1], axis=1),
         jnp.zeros((m, E_DIM - 6), jnp.bfloat16)], axis=1)
    B = jnp.zeros((N_EMBED, E_DIM), jnp.float32)
    B = B.at[0, 0:3].set(1.0).at[N_EMBED // 2, 3:6].set(1.0)
    ws = jnp.full((N_EMBED,), _BIG).at[0].set(0.0).at[N_EMBED // 2].set(0.0)
    zs = jnp.zeros((m,), jnp.float32)
    A, B, zs, ws = lax.optimization_barrier((A, B, zs, ws))
    mm = lax.dot_general(A, B, (((1,), (1,)), ((), ())),
                         preferred_element_type=jnp.float32)
    xr = jnp.maximum(zs[:, None] + ws[None, :] - 2.0 * mm, 0.0)
    pos = jnp.argmin(jnp.sqrt(xr), axis=1)
    return pos != 0


def _codebook_gather(table, idx):
    b = idx.shape[0]
    b_per_w = b // _NW
    mesh = plsc.VectorSubcoreMesh(core_axis_name="c", subcore_axis_name="s")

    @functools.partial(
        pl.kernel,
        mesh=mesh,
        out_type=jax.ShapeDtypeStruct((b, E_DIM), jnp.float32),
        scratch_types=[
            pltpu.VMEM((b_per_w,), jnp.int32),
            pltpu.VMEM((b_per_w, E_DIM), jnp.float32),
            pltpu.SemaphoreType.DMA,
        ],
        compiler_params=pltpu.CompilerParams(use_tc_tiling_on_sc=False),
    )
    def gather_kernel(table_hbm, idx_hbm, out_hbm, idx_v, rows_v, sem):
        wid = lax.axis_index("s") * _SC_CORES + lax.axis_index("c")
        base = wid * b_per_w
        pltpu.sync_copy(idx_hbm.at[pl.ds(base, b_per_w)], idx_v)
        pltpu.async_copy(table_hbm.at[idx_v], rows_v, sem).wait()
        pltpu.sync_copy(rows_v, out_hbm.at[pl.ds(base, b_per_w)])

    return gather_kernel(table, idx)


def kernel(z, embedding_weight):
    z_flat = z.reshape(-1, E_DIM)
    z_sq = jnp.sum(z_flat * z_flat, axis=1, keepdims=True)
    w_sq = jnp.sum(embedding_weight * embedding_weight, axis=1)[None, :]
    wt = embedding_weight.T
    z_bf = z_flat.astype(jnp.bfloat16).astype(jnp.float32)

    idx0, idx1, x0, x1, loss = _distance_halves(
        z_flat.astype(jnp.bfloat16), wt, z_sq, w_sq)
    pick1 = _cross_half_pick(x0.reshape(-1), x1.reshape(-1))
    min_encoding_indices = jnp.where(pick1, idx1.reshape(-1),
                                     idx0.reshape(-1))

    z_q = _codebook_gather(embedding_weight, min_encoding_indices)
    z_q = z_q.reshape(z.shape)

    z_q_st = z + lax.stop_gradient(z_q - z)
    return (z_q_st, loss.reshape(()), min_encoding_indices)


# speedup vs baseline: 1.0347x; 1.0047x over previous
import functools

import jax
import jax.numpy as jnp
from jax import lax
from jax.experimental import pallas as pl
from jax.experimental.pallas import tpu as pltpu
from jax.experimental.pallas import tpu_sc as plsc

N_EMBED = 8192
E_DIM = 64
BETA = 0.25

BM = 512
BN = 1024
NHALF = 4

_SC_CORES = 2
_SC_SUBCORES = 16
_NW = _SC_CORES * _SC_SUBCORES

_BIG = jnp.float32(1e4)


def _vq_body(z_ref, wt_ref, zsq_ref, wsq_ref,
             idx0_ref, idx1_ref, x0_ref, x1_ref, loss_ref,
             mind_ref, arg_ref, x_ref, *, n_rows):
    j = pl.program_id(1)
    nj = pl.num_programs(1)

    @pl.when((j == 0) | (j == NHALF))
    def _init_half():
        mind_ref[...] = jnp.full((BM, 1), jnp.inf, dtype=jnp.float32)
        arg_ref[...] = jnp.zeros((BM, 1), dtype=jnp.int32)
        x_ref[...] = jnp.zeros((BM, 1), dtype=jnp.float32)

    mm = lax.dot_general(z_ref[...], wt_ref[...], (((1,), (0,)), ((), ())),
                         preferred_element_type=jnp.float32)
    d2 = jnp.maximum(zsq_ref[...] + wsq_ref[...] - 2.0 * mm, 0.0)
    dd = jnp.sqrt(d2)

    tmin = jnp.min(dd, axis=1, keepdims=True)
    iota = lax.broadcasted_iota(jnp.int32, (BM, BN), 1)
    tloc = jnp.min(jnp.where(dd == tmin, iota, BN), axis=1, keepdims=True)
    tx = jnp.min(jnp.where(iota == tloc, d2, jnp.inf), axis=1, keepdims=True)
    targ = tloc + j * BN

    upd = tmin < mind_ref[...]
    mind_ref[...] = jnp.where(upd, tmin, mind_ref[...])
    arg_ref[...] = jnp.where(upd, targ, arg_ref[...])
    x_ref[...] = jnp.where(upd, tx, x_ref[...])

    @pl.when(j == NHALF - 1)
    def _half0_out():
        idx0_ref[...] = arg_ref[...]
        x0_ref[...] = x_ref[...]

    @pl.when(j == nj - 1)
    def _half1_out():
        idx1_ref[...] = arg_ref[...]
        x1_ref[...] = x_ref[...]
        part = (jnp.sum(jnp.minimum(x0_ref[...], x_ref[...]))
                * ((1.0 + BETA) / (n_rows * E_DIM)))

        @pl.when(pl.program_id(0) == 0)
        def _zero():
            loss_ref[0, 0] = 0.0

        loss_ref[0, 0] += part


def _distance_halves(z_bf, wt, z_sq, w_sq):
    m = z_bf.shape[0]
    grid = (m // BM, N_EMBED // BN)
    return pl.pallas_call(
        functools.partial(_vq_body, n_rows=m),
        grid=grid,
        in_specs=[
            pl.BlockSpec((BM, E_DIM), lambda i, j: (i, 0)),
            pl.BlockSpec((E_DIM, BN), lambda i, j: (0, j)),
            pl.BlockSpec((BM, 1), lambda i, j: (i, 0)),
            pl.BlockSpec((1, BN), lambda i, j: (0, j)),
        ],
        out_specs=[
            pl.BlockSpec((BM, 1), lambda i, j: (i, 0)),
            pl.BlockSpec((BM, 1), lambda i, j: (i, 0)),
            pl.BlockSpec((BM, 1), lambda i, j: (i, 0)),
            pl.BlockSpec((BM, 1), lambda i, j: (i, 0)),
            pl.BlockSpec(memory_space=pltpu.SMEM, block_shape=(1, 1),
                         index_map=lambda i, j: (0, 0)),
        ],
        out_shape=[
            jax.ShapeDtypeStruct((m, 1), jnp.int32),
            jax.ShapeDtypeStruct((m, 1), jnp.int32),
            jax.ShapeDtypeStruct((m, 1), jnp.float32),
            jax.ShapeDtypeStruct((m, 1), jnp.float32),
            jax.ShapeDtypeStruct((1, 1), jnp.float32),
        ],
        scratch_shapes=[
            pltpu.VMEM((BM, 1), jnp.float32),
            pltpu.VMEM((BM, 1), jnp.int32),
            pltpu.VMEM((BM, 1), jnp.float32),
        ],
    )(z_bf, wt, z_sq, w_sq)


def _rn_bf16(x):
    u = lax.bitcast_convert_type(x, jnp.uint32)
    r = u + jnp.uint32(0x7FFF) + ((u >> 16) & jnp.uint32(1))
    return lax.bitcast_convert_type(r & jnp.uint32(0xFFFF0000), jnp.float32)


def _bf16_split3(x):
    a = _rn_bf16(x)
    r1 = x - a
    b = _rn_bf16(r1)
    c = r1 - b
    return (a.astype(jnp.bfloat16), b.astype(jnp.bfloat16),
            c.astype(jnp.bfloat16))


def _cross_half_pick(x0, x1):
    m = x0.shape[0]
    a0, b0, c0 = _bf16_split3(x0 * jnp.float32(-0.5))
    a1, b1, c1 = _bf16_split3(x1 * jnp.float32(-0.5))
    A = jnp.concatenate(
        [jnp.stack([a0, b0, c0, a1, b1, c1], axis=1),
         jnp.zeros((m, 2), jnp.bfloat16)], axis=1)
    B = jnp.zeros((N_EMBED, 8), jnp.float32)
    B = B.at[0, 0:3].set(1.0).at[N_EMBED // 2, 3:6].set(1.0)
    ws = jnp.full((N_EMBED,), _BIG).at[0].set(0.0).at[N_EMBED // 2].set(0.0)
    zs = jnp.zeros((m,), jnp.float32)
    A, B, zs, ws = lax.optimization_barrier((A, B, zs, ws))
    mm = lax.dot_general(A, B, (((1,), (1,)), ((), ())),
                         preferred_element_type=jnp.float32)
    xr = jnp.maximum(zs[:, None] + ws[None, :] - 2.0 * mm, 0.0)
    pos = jnp.argmin(jnp.sqrt(xr), axis=1)
    return pos != 0


def _codebook_gather(table, idx):
    b = idx.shape[0]
    b_per_w = b // _NW
    mesh = plsc.VectorSubcoreMesh(core_axis_name="c", subcore_axis_name="s")

    @functools.partial(
        pl.kernel,
        mesh=mesh,
        out_type=jax.ShapeDtypeStruct((b, E_DIM), jnp.float32),
        scratch_types=[
            pltpu.VMEM((b_per_w,), jnp.int32),
            pltpu.VMEM((b_per_w, E_DIM), jnp.float32),
            pltpu.SemaphoreType.DMA,
        ],
        compiler_params=pltpu.CompilerParams(use_tc_tiling_on_sc=False),
    )
    def gather_kernel(table_hbm, idx_hbm, out_hbm, idx_v, rows_v, sem):
        wid = lax.axis_index("s") * _SC_CORES + lax.axis_index("c")
        base = wid * b_per_w
        pltpu.sync_copy(idx_hbm.at[pl.ds(base, b_per_w)], idx_v)
        pltpu.async_copy(table_hbm.at[idx_v], rows_v, sem).wait()
        pltpu.sync_copy(rows_v, out_hbm.at[pl.ds(base, b_per_w)])

    return gather_kernel(table, idx)


def kernel(z, embedding_weight):
    z_flat = z.reshape(-1, E_DIM)
    z_sq = jnp.sum(z_flat * z_flat, axis=1, keepdims=True)
    w_sq = jnp.sum(embedding_weight * embedding_weight, axis=1)[None, :]
    wt = embedding_weight.T
    z_bf = z_flat.astype(jnp.bfloat16).astype(jnp.float32)

    idx0, idx1, x0, x1, loss = _distance_halves(
        z_flat.astype(jnp.bfloat16), wt, z_sq, w_sq)
    pick1 = _cross_half_pick(x0.reshape(-1), x1.reshape(-1))
    min_encoding_indices = jnp.where(pick1, idx1.reshape(-1),
                                     idx0.reshape(-1))

    z_q = _codebook_gather(embedding_weight, min_encoding_indices)
    z_q = z_q.reshape(z.shape)

    z_q_st = z + lax.stop_gradient(z_q - z)
    return (z_q_st, loss.reshape(()), min_encoding_indices)


# speedup vs baseline: 1.0867x; 1.0503x over previous
import functools

import jax
import jax.numpy as jnp
from jax import lax
from jax.experimental import pallas as pl
from jax.experimental.pallas import tpu as pltpu
from jax.experimental.pallas import tpu_sc as plsc

N_EMBED = 8192
E_DIM = 64
BETA = 0.25

BM = 1024
BN = 2048
NHALF = 2

_SC_CORES = 2
_SC_SUBCORES = 16
_NW = _SC_CORES * _SC_SUBCORES

_BIG = jnp.float32(1e4)


def _vq_body(z_ref, wt_ref, zsq_ref, wsq_ref,
             idx0_ref, idx1_ref, x0_ref, x1_ref, loss_ref,
             mind_ref, arg_ref, x_ref, *, n_rows):
    j = pl.program_id(1)
    nj = pl.num_programs(1)

    @pl.when((j == 0) | (j == NHALF))
    def _init_half():
        mind_ref[...] = jnp.full((BM, 1), jnp.inf, dtype=jnp.float32)
        arg_ref[...] = jnp.zeros((BM, 1), dtype=jnp.int32)
        x_ref[...] = jnp.zeros((BM, 1), dtype=jnp.float32)

    mm = lax.dot_general(z_ref[...], wt_ref[...], (((1,), (0,)), ((), ())),
                         preferred_element_type=jnp.float32)
    d2 = jnp.maximum(zsq_ref[...] + wsq_ref[...] - 2.0 * mm, 0.0)
    dd = jnp.sqrt(d2)

    tmin = jnp.min(dd, axis=1, keepdims=True)
    iota = lax.broadcasted_iota(jnp.int32, (BM, BN), 1)
    tloc = jnp.min(jnp.where(dd == tmin, iota, BN), axis=1, keepdims=True)
    tx = jnp.min(jnp.where(iota == tloc, d2, jnp.inf), axis=1, keepdims=True)
    targ = tloc + j * BN

    upd = tmin < mind_ref[...]
    mind_ref[...] = jnp.where(upd, tmin, mind_ref[...])
    arg_ref[...] = jnp.where(upd, targ, arg_ref[...])
    x_ref[...] = jnp.where(upd, tx, x_ref[...])

    @pl.when(j == NHALF - 1)
    def _half0_out():
        idx0_ref[...] = arg_ref[...]
        x0_ref[...] = x_ref[...]

    @pl.when(j == nj - 1)
    def _half1_out():
        idx1_ref[...] = arg_ref[...]
        x1_ref[...] = x_ref[...]
        part = (jnp.sum(jnp.minimum(x0_ref[...], x_ref[...]))
                * ((1.0 + BETA) / (n_rows * E_DIM)))

        @pl.when(pl.program_id(0) == 0)
        def _zero():
            loss_ref[0, 0] = 0.0

        loss_ref[0, 0] += part


def _distance_halves(z_bf, wt, z_sq, w_sq):
    m = z_bf.shape[0]
    grid = (m // BM, N_EMBED // BN)
    return pl.pallas_call(
        functools.partial(_vq_body, n_rows=m),
        grid=grid,
        in_specs=[
            pl.BlockSpec((BM, E_DIM), lambda i, j: (i, 0)),
            pl.BlockSpec((E_DIM, BN), lambda i, j: (0, j)),
            pl.BlockSpec((BM, 1), lambda i, j: (i, 0)),
            pl.BlockSpec((1, BN), lambda i, j: (0, j)),
        ],
        out_specs=[
            pl.BlockSpec((BM, 1), lambda i, j: (i, 0)),
            pl.BlockSpec((BM, 1), lambda i, j: (i, 0)),
            pl.BlockSpec((BM, 1), lambda i, j: (i, 0)),
            pl.BlockSpec((BM, 1), lambda i, j: (i, 0)),
            pl.BlockSpec(memory_space=pltpu.SMEM, block_shape=(1, 1),
                         index_map=lambda i, j: (0, 0)),
        ],
        out_shape=[
            jax.ShapeDtypeStruct((m, 1), jnp.int32),
            jax.ShapeDtypeStruct((m, 1), jnp.int32),
            jax.ShapeDtypeStruct((m, 1), jnp.float32),
            jax.ShapeDtypeStruct((m, 1), jnp.float32),
            jax.ShapeDtypeStruct((1, 1), jnp.float32),
        ],
        scratch_shapes=[
            pltpu.VMEM((BM, 1), jnp.float32),
            pltpu.VMEM((BM, 1), jnp.int32),
            pltpu.VMEM((BM, 1), jnp.float32),
        ],
    )(z_bf, wt, z_sq, w_sq)


def _rn_bf16(x):
    u = lax.bitcast_convert_type(x, jnp.uint32)
    r = u + jnp.uint32(0x7FFF) + ((u >> 16) & jnp.uint32(1))
    return lax.bitcast_convert_type(r & jnp.uint32(0xFFFF0000), jnp.float32)


def _bf16_split3(x):
    a = _rn_bf16(x)
    r1 = x - a
    b = _rn_bf16(r1)
    c = r1 - b
    return (a.astype(jnp.bfloat16), b.astype(jnp.bfloat16),
            c.astype(jnp.bfloat16))


def _cross_half_pick(x0, x1):
    m = x0.shape[0]
    a0, b0, c0 = _bf16_split3(x0 * jnp.float32(-0.5))
    a1, b1, c1 = _bf16_split3(x1 * jnp.float32(-0.5))
    A = jnp.concatenate(
        [jnp.stack([a0, b0, c0, a1, b1, c1], axis=1),
         jnp.zeros((m, 2), jnp.bfloat16)], axis=1)
    B = jnp.zeros((N_EMBED, 8), jnp.float32)
    B = B.at[0, 0:3].set(1.0).at[N_EMBED // 2, 3:6].set(1.0)
    ws = jnp.full((N_EMBED,), _BIG).at[0].set(0.0).at[N_EMBED // 2].set(0.0)
    zs = jnp.zeros((m,), jnp.float32)
    A, B, zs, ws = lax.optimization_barrier((A, B, zs, ws))
    mm = lax.dot_general(A, B, (((1,), (1,)), ((), ())),
                         preferred_element_type=jnp.float32)
    xr = jnp.maximum(zs[:, None] + ws[None, :] - 2.0 * mm, 0.0)
    pos = jnp.argmin(jnp.sqrt(xr), axis=1)
    return pos != 0


def _codebook_gather(table, idx):
    b = idx.shape[0]
    b_per_w = b // _NW
    mesh = plsc.VectorSubcoreMesh(core_axis_name="c", subcore_axis_name="s")

    @functools.partial(
        pl.kernel,
        mesh=mesh,
        out_type=jax.ShapeDtypeStruct((b, E_DIM), jnp.float32),
        scratch_types=[
            pltpu.VMEM((b_per_w,), jnp.int32),
            pltpu.VMEM((b_per_w, E_DIM), jnp.float32),
            pltpu.SemaphoreType.DMA,
        ],
        compiler_params=pltpu.CompilerParams(use_tc_tiling_on_sc=False),
    )
    def gather_kernel(table_hbm, idx_hbm, out_hbm, idx_v, rows_v, sem):
        wid = lax.axis_index("s") * _SC_CORES + lax.axis_index("c")
        base = wid * b_per_w
        pltpu.sync_copy(idx_hbm.at[pl.ds(base, b_per_w)], idx_v)
        pltpu.async_copy(table_hbm.at[idx_v], rows_v, sem).wait()
        pltpu.sync_copy(rows_v, out_hbm.at[pl.ds(base, b_per_w)])

    return gather_kernel(table, idx)


def kernel(z, embedding_weight):
    z_flat = z.reshape(-1, E_DIM)
    z_sq = jnp.sum(z_flat * z_flat, axis=1, keepdims=True)
    w_sq = jnp.sum(embedding_weight * embedding_weight, axis=1)[None, :]
    wt = embedding_weight.T
    z_bf = z_flat.astype(jnp.bfloat16).astype(jnp.float32)

    idx0, idx1, x0, x1, loss = _distance_halves(
        z_flat.astype(jnp.bfloat16), wt, z_sq, w_sq)
    pick1 = _cross_half_pick(x0.reshape(-1), x1.reshape(-1))
    min_encoding_indices = jnp.where(pick1, idx1.reshape(-1),
                                     idx0.reshape(-1))

    z_q = _codebook_gather(embedding_weight, min_encoding_indices)
    z_q = z_q.reshape(z.shape)

    z_q_st = z + lax.stop_gradient(z_q - z)
    return (z_q_st, loss.reshape(()), min_encoding_indices)
